# baseline (speedup 1.0000x reference)
import jax
import jax.numpy as jnp
from jax import lax
from jax.experimental import pallas as pl
from jax.experimental.pallas import tpu as pltpu
from jax.experimental.pallas import tpu_sc as plsc

D = 64
NROWS = 1_000_000
NPAIR = NROWS // 2
NW = 32
FULL_CHUNKS = NROWS // 128
TAIL_COLS = NROWS - FULL_CHUNKS * 128
K1_STEPS = FULL_CHUNKS // NW + 1

_mesh = plsc.VectorSubcoreMesh(core_axis_name="core", subcore_axis_name="subcore")
_IOTA = lambda: lax.iota(jnp.int32, 16)


def _k1_relayout(tableT, tailP):
    @pl.kernel(
        out_type=jax.ShapeDtypeStruct((NPAIR, 128), jnp.float32),
        mesh=_mesh,
        compiler_params=pltpu.CompilerParams(needs_layout_passes=False),
        scratch_types=[
            pltpu.VMEM((64, 128), jnp.float32),
            pltpu.VMEM((64, 128), jnp.float32),
            pltpu.VMEM((64, 128), jnp.float32),
            pltpu.SemaphoreType.DMA,
            pltpu.SemaphoreType.DMA,
        ],
    )
    def k1(tt, tp, tr, tb0, tb1, ob, s0, s1):
        w = lax.axis_index("subcore") * 2 + lax.axis_index("core")
        tbufs, sems = (tb0, tb1), (s0, s1)
        iota = _IOTA()

        def full_copy(c, b):
            return pltpu.make_async_copy(
                tt.at[:, pl.ds(c * 128, 128)], tbufs[b], sems[b])

        def issue(c, b):
            @pl.when(c < FULL_CHUNKS)
            def _():
                full_copy(c, b).start()

        def consume(c, b):
            @pl.when(c < FULL_CHUNKS)
            def _():
                full_copy(c, b).wait()
                tb = tbufs[b]

                @pl.loop(0, 128)
                def _(r):
                    half = (r % 2) * D
                    for d0 in range(0, D, 16):
                        v = plsc.load_gather(tb, [iota + d0, iota * 0 + r])
                        ob[r // 2, pl.ds(half + d0, 16)] = v

                pltpu.sync_copy(ob, tr.at[pl.ds(c * D, D), :])

        @pl.when(w == 0)
        def _():
            pltpu.sync_copy(tp, ob.at[pl.ds(0, TAIL_COLS // 2), :])
            pltpu.sync_copy(
                ob.at[pl.ds(0, TAIL_COLS // 2), :],
                tr.at[pl.ds(FULL_CHUNKS * D, TAIL_COLS // 2), :])

        issue(w, 0)

        @pl.loop(0, K1_STEPS + 1, step=2)
        def _(t0):
            for b in range(2):
                t = t0 + b
                c = w + NW * t
                issue(w + NW * (t + 1), 1 - b)
                consume(c, b)

    return k1(tableT, tailP)


def _k2_gather(tableR, xT):
    n_units = 200 * 32
    per_tec = n_units // NW

    @pl.kernel(
        out_type=jax.ShapeDtypeStruct((200, D, 4096), jnp.float32),
        mesh=_mesh,
        compiler_params=pltpu.CompilerParams(needs_layout_passes=False),
        scratch_types=[
            pltpu.VMEM((128,), jnp.int32),
            pltpu.VMEM((128,), jnp.int32),
            pltpu.VMEM((128,), jnp.int32),
            pltpu.VMEM((128,), jnp.int32),
            pltpu.VMEM((128,), jnp.int32),
            pltpu.VMEM((128,), jnp.int32),
            pltpu.VMEM((128, 128), jnp.float32),
            pltpu.VMEM((128, 128), jnp.float32),
            pltpu.VMEM((D, 128), jnp.float32),
            pltpu.SemaphoreType.DMA,
            pltpu.SemaphoreType.DMA,
        ],
    )
    def k2(tr, xt, out, i0_, i1_, q0_, q1_, a0_, a1_, g0, g1, ob, s0, s1):
        w = lax.axis_index("subcore") * 2 + lax.axis_index("core")
        ibufs, qbufs, abufs = (i0_, i1_), (q0_, q1_), (a0_, a1_)
        gbufs, sems = (g0, g1), (s0, s1)
        iota = _IOTA()

        def gather_copy(b):
            return pltpu.make_async_copy(tr.at[qbufs[b]], gbufs[b], sems[b])

        def prep(t, b):
            u = w * per_tec + t
            j = u // 32
            i0 = (u % 32) * 128
            pltpu.sync_copy(xt.at[j, pl.ds(i0, 128)], ibufs[b])
            for k in range(0, 128, 16):
                iv = ibufs[b][pl.ds(k, 16)]
                qbufs[b][pl.ds(k, 16)] = lax.shift_right_logical(iv, 1)
                abufs[b][pl.ds(k, 16)] = (iv & 1) * D
            gather_copy(b).start()

        def consume(t, b):
            u = w * per_tec + t
            j = u // 32
            i0 = (u % 32) * 128
            gather_copy(b).wait()
            gb, ab = gbufs[b], abufs[b]

            @pl.loop(0, D)
            def _(d):
                for lg in range(0, 128, 16):
                    col = ab[pl.ds(lg, 16)] + d
                    v = plsc.load_gather(gb, [iota + lg, col])
                    ob[d, pl.ds(lg, 16)] = v

            pltpu.sync_copy(ob, out.at[j, :, pl.ds(i0, 128)])

        prep(0, 0)

        @pl.loop(0, per_tec, step=2)
        def _(t0):
            for b in range(2):
                t = t0 + b

                @pl.when(t + 1 < per_tec)
                def _():
                    prep(t + 1, 1 - b)

                consume(t, b)

    return k2(tableR, xT)


def kernel(x, table):
    tableT = table.T
    xT = x.T.astype(jnp.int32)
    tailP = table[NROWS - TAIL_COLS:, :].reshape(TAIL_COLS // 2, 128)
    tableR = _k1_relayout(tableT, tailP)
    outT = _k2_gather(tableR, xT)
    return outT.transpose(2, 0, 1)


# speedup vs baseline: 2.9274x; 2.9274x over previous
import jax
import jax.numpy as jnp
from jax import lax
from jax.experimental import pallas as pl
from jax.experimental.pallas import tpu as pltpu
from jax.experimental.pallas import tpu_sc as plsc

D = 64
B0, B1 = 4096, 200
NW = 32
PER_W = B0 // NW * B1
UNIT = 256
N_UNITS = PER_W // UNIT
NSLOT = 4

_mesh = plsc.VectorSubcoreMesh(core_axis_name="core", subcore_axis_name="subcore")


def _gather(table, xT):
    @pl.kernel(
        out_type=jax.ShapeDtypeStruct((B0 * B1, D), jnp.float32),
        mesh=_mesh,
        compiler_params=pltpu.CompilerParams(
            use_tc_tiling_on_sc=False, needs_layout_passes=False),
        scratch_types=[
            pltpu.VMEM((B1, 128), jnp.int32),
            pltpu.VMEM((PER_W,), jnp.int32),
            pltpu.VMEM((UNIT, D), jnp.float32),
            pltpu.VMEM((UNIT, D), jnp.float32),
            pltpu.VMEM((UNIT, D), jnp.float32),
            pltpu.VMEM((UNIT, D), jnp.float32),
            pltpu.SemaphoreType.DMA,
            pltpu.SemaphoreType.DMA,
            pltpu.SemaphoreType.DMA,
            pltpu.SemaphoreType.DMA,
            pltpu.SemaphoreType.DMA,
            pltpu.SemaphoreType.DMA,
            pltpu.SemaphoreType.DMA,
            pltpu.SemaphoreType.DMA,
        ],
    )
    def k(tab, xt, out, xtb, xfl, g0, g1, g2, g3,
          sg0, sg1, sg2, sg3, so0, so1, so2, so3):
        w = lax.axis_index("subcore") * 2 + lax.axis_index("core")
        gbufs = (g0, g1, g2, g3)
        sgs = (sg0, sg1, sg2, sg3)
        sos = (so0, so1, so2, so3)
        iota = lax.iota(jnp.int32, 16)
        i0 = w * 128
        base = w * PER_W

        pltpu.sync_copy(xt.at[:, pl.ds(i0, 128)], xtb)
        j_chunks = list(range(0, B1 - 16, 16)) + [B1 - 16]

        @pl.loop(0, 128)
        def _(ii):
            ci = iota * 0 + ii
            for j0 in j_chunks:
                v = plsc.load_gather(xtb, [iota + j0, ci])
                xfl[pl.ds(ii * B1 + j0, 16)] = v

        def gather_copy(u, b):
            return pltpu.make_async_copy(
                tab.at[xfl.at[pl.ds(u * UNIT, UNIT)]], gbufs[b], sgs[b])

        def write_copy(u, b):
            return pltpu.make_async_copy(
                gbufs[b], out.at[pl.ds(base + u * UNIT, UNIT), :], sos[b])

        gather_copy(0, 0).start()
        gather_copy(1, 1).start()

        @pl.loop(0, N_UNITS, step=NSLOT)
        def _(u0):
            for db in range(NSLOT):
                u = u0 + db
                b = db % NSLOT
                nb = (db + 2) % NSLOT

                @pl.when(u >= 2)
                def _():
                    write_copy(u - 2, nb).wait()

                @pl.when(u + 2 < N_UNITS)
                def _():
                    gather_copy(u + 2, nb).start()

                gather_copy(u, b).wait()
                write_copy(u, b).start()

        write_copy(N_UNITS - 2, (N_UNITS - 2) % NSLOT).wait()
        write_copy(N_UNITS - 1, (N_UNITS - 1) % NSLOT).wait()

    return k(table, xT)


def kernel(x, table):
    xT = x.T.astype(jnp.int32)
    out = _gather(table, xT)
    return out.reshape(B0, B1, D)


# speedup vs baseline: 2.9409x; 1.0046x over previous
import jax
import jax.numpy as jnp
from jax import lax
from jax.experimental import pallas as pl
from jax.experimental.pallas import tpu as pltpu
from jax.experimental.pallas import tpu_sc as plsc

D = 64
B0, B1 = 4096, 200
NW = 32
PER_W = B0 // NW * B1
UNIT = 256
N_UNITS = PER_W // UNIT
NSLOT = 4

_mesh = plsc.VectorSubcoreMesh(core_axis_name="core", subcore_axis_name="subcore")


def _gather(table, xT):
    @pl.kernel(
        out_type=jax.ShapeDtypeStruct((B0 * B1, D), jnp.float32),
        mesh=_mesh,
        compiler_params=pltpu.CompilerParams(
            use_tc_tiling_on_sc=False, needs_layout_passes=False),
        scratch_types=[
            pltpu.VMEM((B1 // 8, 8, 128), jnp.int32),
            pltpu.VMEM((PER_W,), jnp.int32),
            pltpu.VMEM((UNIT, D), jnp.float32),
            pltpu.VMEM((UNIT, D), jnp.float32),
            pltpu.VMEM((UNIT, D), jnp.float32),
            pltpu.VMEM((UNIT, D), jnp.float32),
            pltpu.SemaphoreType.DMA,
            pltpu.SemaphoreType.DMA,
            pltpu.SemaphoreType.DMA,
            pltpu.SemaphoreType.DMA,
            pltpu.SemaphoreType.DMA,
            pltpu.SemaphoreType.DMA,
            pltpu.SemaphoreType.DMA,
            pltpu.SemaphoreType.DMA,
        ],
    )
    def k(tab, xt, out, xtb, xfl, g0, g1, g2, g3,
          sg0, sg1, sg2, sg3, so0, so1, so2, so3):
        w = lax.axis_index("subcore") * 2 + lax.axis_index("core")
        gbufs = (g0, g1, g2, g3)
        sgs = (sg0, sg1, sg2, sg3)
        sos = (so0, so1, so2, so3)
        iota = lax.iota(jnp.int32, 16)
        i0 = w * 128
        base = w * PER_W

        pltpu.sync_copy(xt.at[:, i0 // 128, :, :], xtb)
        j_chunks = list(range(0, B1 - 16, 16)) + [B1 - 16]
        tj_off = lax.shift_right_logical(iota, 3)
        s_off = iota & 7

        @pl.loop(0, 128)
        def _(ii):
            ci = iota * 0 + ii
            for j0 in j_chunks:
                v = plsc.load_gather(xtb, [tj_off + j0 // 8, s_off, ci])
                xfl[pl.ds(ii * B1 + j0, 16)] = v

        def gather_copy(u, b):
            return pltpu.make_async_copy(
                tab.at[xfl.at[pl.ds(u * UNIT, UNIT)]], gbufs[b], sgs[b])

        def write_copy(u, b):
            return pltpu.make_async_copy(
                gbufs[b], out.at[pl.ds(base + u * UNIT, UNIT), :], sos[b])

        gather_copy(0, 0).start()
        gather_copy(1, 1).start()

        @pl.loop(0, N_UNITS, step=NSLOT)
        def _(u0):
            for db in range(NSLOT):
                u = u0 + db
                b = db % NSLOT
                nb = (db + 2) % NSLOT

                @pl.when(u >= 2)
                def _():
                    write_copy(u - 2, nb).wait()

                @pl.when(u + 2 < N_UNITS)
                def _():
                    gather_copy(u + 2, nb).start()

                gather_copy(u, b).wait()
                write_copy(u, b).start()

        write_copy(N_UNITS - 2, (N_UNITS - 2) % NSLOT).wait()
        write_copy(N_UNITS - 1, (N_UNITS - 1) % NSLOT).wait()

    return k(table, xT)


def kernel(x, table):
    xv = x.astype(jnp.int32).T.reshape(B1 // 8, 8, B0 // 128, 128)
    xv = xv.transpose(0, 2, 1, 3)
    out = _gather(table, xv)
    return out.reshape(B0, B1, D)
